# Initial kernel scaffold; baseline (speedup 1.0000x reference)
import jax
import jax.numpy as jnp
from jax.experimental import pallas as pl


def kernel(x, edge_index, W1, b1, W2, b2, W3, b3):
    raise NotImplementedError("write your pallas kernel here")


# speedup vs baseline: 24.5234x; 24.5234x over previous
import functools

import jax
import jax.numpy as jnp
from jax import lax
from jax.experimental import pallas as pl
from jax.experimental.pallas import tpu as pltpu
from jax.experimental.pallas import tpu_sc as plsc

_N = 10000
_NPAD = 10240
_E = 320000
_LANES = 128
_NTILES = 32
_CPT = 80
_EPAD = _NTILES * _CPT * _LANES
_RPT = _NPAD // 16
_BLK = 256
_GRID = _NPAD // _BLK


def _sc_mesh():
    return plsc.VectorSubcoreMesh(
        core_axis_name="c", subcore_axis_name="s", num_cores=2, num_subcores=16
    )


def _make_agg(F):
    @functools.partial(
        pl.kernel,
        out_type=jax.ShapeDtypeStruct((2, _NPAD, F), jnp.float32),
        mesh=_sc_mesh(),
        compiler_params=pltpu.CompilerParams(use_tc_tiling_on_sc=False),
        scratch_types=[
            pltpu.VMEM((_CPT, _LANES), jnp.int32),
            pltpu.VMEM((_CPT, _LANES), jnp.int32),
            pltpu.VMEM((_LANES, F), jnp.float32),
            pltpu.VMEM_SHARED((_NPAD, F), jnp.float32),
            pltpu.SemaphoreType.DMA,
        ],
    )
    def agg(src_hbm, dst_hbm, g_hbm, z_hbm, out_hbm, src_v, dst_v, rows_v, acc_sh, sem):
        c = lax.axis_index("c")
        s = lax.axis_index("s")
        wid = c * 16 + s
        row0 = s * _RPT
        pltpu.sync_copy(z_hbm.at[pl.ds(row0, _RPT)], acc_sh.at[pl.ds(row0, _RPT)])
        pltpu.sync_copy(src_hbm.at[pl.ds(wid * _CPT, _CPT)], src_v)
        pltpu.sync_copy(dst_hbm.at[pl.ds(wid * _CPT, _CPT)], dst_v)
        plsc.subcore_barrier()

        def body(i, carry):
            pltpu.async_copy(g_hbm.at[src_v.at[i]], rows_v, sem).wait()
            pltpu.sync_copy(rows_v, acc_sh.at[dst_v.at[i]], add=True)
            return carry

        lax.fori_loop(0, _CPT, body, 0)
        plsc.subcore_barrier()
        pltpu.sync_copy(acc_sh.at[pl.ds(row0, _RPT)], out_hbm.at[c, pl.ds(row0, _RPT)])

    return agg


def _make_deg():
    @functools.partial(
        pl.kernel,
        out_type=jax.ShapeDtypeStruct((2 * _NPAD,), jnp.float32),
        mesh=_sc_mesh(),
        scratch_types=[
            pltpu.VMEM((_CPT, _LANES), jnp.int32),
            pltpu.VMEM((_LANES,), jnp.float32),
            pltpu.VMEM_SHARED((_NPAD,), jnp.float32),
        ],
    )
    def deg(dst_hbm, z_hbm, out_hbm, dst_v, ones_v, acc_sh):
        c = lax.axis_index("c")
        s = lax.axis_index("s")
        wid = c * 16 + s
        row0 = s * _RPT
        pltpu.sync_copy(z_hbm.at[pl.ds(row0, _RPT)], acc_sh.at[pl.ds(row0, _RPT)])
        pltpu.sync_copy(dst_hbm.at[pl.ds(wid * _CPT, _CPT)], dst_v)
        one = jnp.ones((16,), jnp.float32)
        for j in range(_LANES // 16):
            ones_v[pl.ds(j * 16, 16)] = one
        plsc.subcore_barrier()

        def body(i, carry):
            pltpu.sync_copy(ones_v, acc_sh.at[dst_v.at[i]], add=True)
            return carry

        lax.fori_loop(0, _CPT, body, 0)
        plsc.subcore_barrier()
        pltpu.sync_copy(
            acc_sh.at[pl.ds(row0, _RPT)], out_hbm.at[pl.ds(c * _NPAD + row0, _RPT)]
        )

    return deg


def _dinv_of(d0, d1):
    return lax.rsqrt(jnp.maximum(d0 + d1 + 1.0, 1e-12))


def _tc_first(xp, W, d0, d1):
    Fin, Fout = W.shape

    def body(x_ref, w_ref, d0_ref, d1_ref, o_ref):
        dinv = _dinv_of(d0_ref[...], d1_ref[...])
        o_ref[...] = dinv * jnp.dot(
            x_ref[...], w_ref[...], preferred_element_type=jnp.float32
        )

    return pl.pallas_call(
        body,
        grid=(_GRID,),
        in_specs=[
            pl.BlockSpec((_BLK, Fin), lambda i: (i, 0)),
            pl.BlockSpec((Fin, Fout), lambda i: (0, 0)),
            pl.BlockSpec((_BLK, 1), lambda i: (i, 0)),
            pl.BlockSpec((_BLK, 1), lambda i: (i, 0)),
        ],
        out_specs=pl.BlockSpec((_BLK, Fout), lambda i: (i, 0)),
        out_shape=jax.ShapeDtypeStruct((_NPAD, Fout), jnp.float32),
    )(xp, W, d0, d1)


def _tc_mid(a0, a1, g, d0, d1, b, W):
    Fin, Fout = W.shape

    def body(a0_ref, a1_ref, g_ref, d0_ref, d1_ref, b_ref, w_ref, o_ref):
        dinv = _dinv_of(d0_ref[...], d1_ref[...])
        h = jnp.maximum(dinv * (a0_ref[...] + a1_ref[...] + g_ref[...]) + b_ref[...], 0.0)
        o_ref[...] = dinv * jnp.dot(h, w_ref[...], preferred_element_type=jnp.float32)

    return pl.pallas_call(
        body,
        grid=(_GRID,),
        in_specs=[
            pl.BlockSpec((_BLK, Fin), lambda i: (i, 0)),
            pl.BlockSpec((_BLK, Fin), lambda i: (i, 0)),
            pl.BlockSpec((_BLK, Fin), lambda i: (i, 0)),
            pl.BlockSpec((_BLK, 1), lambda i: (i, 0)),
            pl.BlockSpec((_BLK, 1), lambda i: (i, 0)),
            pl.BlockSpec((1, Fin), lambda i: (0, 0)),
            pl.BlockSpec((Fin, Fout), lambda i: (0, 0)),
        ],
        out_specs=pl.BlockSpec((_BLK, Fout), lambda i: (i, 0)),
        out_shape=jax.ShapeDtypeStruct((_NPAD, Fout), jnp.float32),
    )(a0, a1, g, d0, d1, b, W)


def _tc_last(a0, a1, g, d0, d1, b):
    F = g.shape[1]

    def body(a0_ref, a1_ref, g_ref, d0_ref, d1_ref, b_ref, o_ref):
        dinv = _dinv_of(d0_ref[...], d1_ref[...])
        o = (dinv * (a0_ref[...] + a1_ref[...] + g_ref[...]) + b_ref[...])[:, :2]
        m = jnp.max(o, axis=1, keepdims=True)
        e = jnp.exp(o - m)
        o_ref[...] = o - m - jnp.log(jnp.sum(e, axis=1, keepdims=True))

    return pl.pallas_call(
        body,
        grid=(_GRID,),
        in_specs=[
            pl.BlockSpec((_BLK, F), lambda i: (i, 0)),
            pl.BlockSpec((_BLK, F), lambda i: (i, 0)),
            pl.BlockSpec((_BLK, F), lambda i: (i, 0)),
            pl.BlockSpec((_BLK, 1), lambda i: (i, 0)),
            pl.BlockSpec((_BLK, 1), lambda i: (i, 0)),
            pl.BlockSpec((1, F), lambda i: (0, 0)),
        ],
        out_specs=pl.BlockSpec((_BLK, 2), lambda i: (i, 0)),
        out_shape=jax.ShapeDtypeStruct((_NPAD, 2), jnp.float32),
    )(a0, a1, g, d0, d1, b)


def kernel(x, edge_index, W1, b1, W2, b2, W3, b3):
    xp = jnp.pad(x, ((0, _NPAD - _N), (0, 0)))
    src = edge_index[0].astype(jnp.int32)
    dst = edge_index[1].astype(jnp.int32)
    pad = _EPAD - _E
    it = jnp.arange(pad, dtype=jnp.int32)
    srcp = jnp.concatenate([src, it % _N]).reshape(-1, _LANES)
    dstp = jnp.concatenate([dst, _N + it % (_NPAD - _N)]).reshape(-1, _LANES)

    degp = _make_deg()(dstp, jnp.zeros((_NPAD,), jnp.float32)).reshape(2, _NPAD)
    d0 = degp[0].reshape(_NPAD, 1)
    d1 = degp[1].reshape(_NPAD, 1)

    g1 = _tc_first(xp, W1, d0, d1)
    a1 = _make_agg(32)(srcp, dstp, g1, jnp.zeros((_NPAD, 32), jnp.float32))
    g2 = _tc_mid(a1[0], a1[1], g1, d0, d1, b1.reshape(1, -1), W2)
    a2 = _make_agg(16)(srcp, dstp, g2, jnp.zeros((_NPAD, 16), jnp.float32))
    W3p = jnp.pad(W3, ((0, 0), (0, 6)))
    b3p = jnp.pad(b3, (0, 6)).reshape(1, 8)
    g3 = _tc_mid(a2[0], a2[1], g2, d0, d1, b2.reshape(1, -1), W3p)
    a3 = _make_agg(8)(srcp, dstp, g3, jnp.zeros((_NPAD, 8), jnp.float32))
    o = _tc_last(a3[0], a3[1], g3, d0, d1, b3p)
    return o[:_N]


# speedup vs baseline: 37.6935x; 1.5370x over previous
import functools

import jax
import jax.numpy as jnp
from jax import lax
from jax.experimental import pallas as pl
from jax.experimental.pallas import tpu as pltpu
from jax.experimental.pallas import tpu_sc as plsc

_N = 10000
_NPAD = 10240
_E = 320000
_LANES = 128
_NTILES = 32
_CPT = 80
_EPAD = _NTILES * _CPT * _LANES
_RPT = _NPAD // 16
_BLK = 256
_GRID = _NPAD // _BLK


def _sc_mesh():
    return plsc.VectorSubcoreMesh(
        core_axis_name="c", subcore_axis_name="s", num_cores=2, num_subcores=16
    )


def _make_agg(F):
    K = 8
    G = _CPT // K

    @functools.partial(
        pl.kernel,
        out_type=jax.ShapeDtypeStruct((2, _NPAD, F), jnp.float32),
        mesh=_sc_mesh(),
        compiler_params=pltpu.CompilerParams(use_tc_tiling_on_sc=False),
        scratch_types=[
            pltpu.VMEM((_CPT, _LANES), jnp.int32),
            pltpu.VMEM((_CPT, _LANES), jnp.int32),
            pltpu.VMEM((K * _LANES, F), jnp.float32),
            pltpu.VMEM((K * _LANES, F), jnp.float32),
            pltpu.VMEM_SHARED((_NPAD, F), jnp.float32),
            pltpu.SemaphoreType.DMA,
            pltpu.SemaphoreType.DMA,
            pltpu.SemaphoreType.DMA,
            pltpu.SemaphoreType.DMA,
        ],
    )
    def agg(src_hbm, dst_hbm, g_hbm, z_hbm, out_hbm, src_v, dst_v, buf_a, buf_b,
            acc_sh, gsem_a, gsem_b, ssem_a, ssem_b):
        c = lax.axis_index("c")
        s = lax.axis_index("s")
        wid = c * 16 + s
        row0 = s * _RPT
        pltpu.sync_copy(z_hbm.at[pl.ds(row0, _RPT)], acc_sh.at[pl.ds(row0, _RPT)])
        pltpu.sync_copy(src_hbm.at[pl.ds(wid * _CPT, _CPT)], src_v)
        pltpu.sync_copy(dst_hbm.at[pl.ds(wid * _CPT, _CPT)], dst_v)
        plsc.subcore_barrier()

        def gathers(g, buf, sem, issue):
            for b in range(K):
                d = pltpu.make_async_copy(
                    g_hbm.at[src_v.at[g * K + b]], buf.at[pl.ds(b * _LANES, _LANES)], sem
                )
                d.start() if issue else d.wait()

        def scatters(g, buf, sem, issue):
            for b in range(K):
                if issue:
                    pltpu.async_copy(
                        buf.at[pl.ds(b * _LANES, _LANES)],
                        acc_sh.at[dst_v.at[g * K + b]], sem, add=True,
                    )
                else:
                    pltpu.make_async_copy(
                        buf.at[pl.ds(b * _LANES, _LANES)],
                        acc_sh.at[dst_v.at[g * K + b]], sem,
                    ).wait()

        def group(g, buf, gsem, ssem, prefetch):
            gathers(g, buf, gsem, False)
            scatters(g, buf, ssem, True)
            scatters(g, buf, ssem, False)
            if prefetch:
                gathers(g + 2, buf, gsem, True)

        gathers(0, buf_a, gsem_a, True)
        gathers(1, buf_b, gsem_b, True)

        def pair(gp, carry):
            group(2 * gp, buf_a, gsem_a, ssem_a, True)
            group(2 * gp + 1, buf_b, gsem_b, ssem_b, True)
            return carry

        lax.fori_loop(0, G // 2 - 1, pair, 0)
        group(G - 2, buf_a, gsem_a, ssem_a, False)
        group(G - 1, buf_b, gsem_b, ssem_b, False)
        plsc.subcore_barrier()
        pltpu.sync_copy(acc_sh.at[pl.ds(row0, _RPT)], out_hbm.at[c, pl.ds(row0, _RPT)])

    return agg


def _make_deg():
    @functools.partial(
        pl.kernel,
        out_type=jax.ShapeDtypeStruct((2 * _NPAD,), jnp.float32),
        mesh=_sc_mesh(),
        scratch_types=[
            pltpu.VMEM((_CPT, _LANES), jnp.int32),
            pltpu.VMEM((_LANES,), jnp.float32),
            pltpu.VMEM_SHARED((_NPAD,), jnp.float32),
            pltpu.SemaphoreType.DMA,
        ],
    )
    def deg(dst_hbm, z_hbm, out_hbm, dst_v, ones_v, acc_sh, sem):
        c = lax.axis_index("c")
        s = lax.axis_index("s")
        wid = c * 16 + s
        row0 = s * _RPT
        pltpu.sync_copy(z_hbm.at[pl.ds(row0, _RPT)], acc_sh.at[pl.ds(row0, _RPT)])
        pltpu.sync_copy(dst_hbm.at[pl.ds(wid * _CPT, _CPT)], dst_v)
        one = jnp.ones((16,), jnp.float32)
        for j in range(_LANES // 16):
            ones_v[pl.ds(j * 16, 16)] = one
        plsc.subcore_barrier()

        def body(i, carry):
            pltpu.async_copy(ones_v, acc_sh.at[dst_v.at[i]], sem, add=True)
            return carry

        lax.fori_loop(0, _CPT, body, 0)

        def drain(i, carry):
            pltpu.make_async_copy(ones_v, acc_sh.at[dst_v.at[i]], sem).wait()
            return carry

        lax.fori_loop(0, _CPT, drain, 0)
        plsc.subcore_barrier()
        pltpu.sync_copy(
            acc_sh.at[pl.ds(row0, _RPT)], out_hbm.at[pl.ds(c * _NPAD + row0, _RPT)]
        )

    return deg


def _dinv_of(d0, d1):
    return lax.rsqrt(jnp.maximum(d0 + d1 + 1.0, 1e-12))


def _tc_first(xp, W, d0, d1):
    Fin, Fout = W.shape

    def body(x_ref, w_ref, d0_ref, d1_ref, o_ref):
        dinv = _dinv_of(d0_ref[...], d1_ref[...])
        o_ref[...] = dinv * jnp.dot(
            x_ref[...], w_ref[...], preferred_element_type=jnp.float32
        )

    return pl.pallas_call(
        body,
        grid=(_GRID,),
        in_specs=[
            pl.BlockSpec((_BLK, Fin), lambda i: (i, 0)),
            pl.BlockSpec((Fin, Fout), lambda i: (0, 0)),
            pl.BlockSpec((_BLK, 1), lambda i: (i, 0)),
            pl.BlockSpec((_BLK, 1), lambda i: (i, 0)),
        ],
        out_specs=pl.BlockSpec((_BLK, Fout), lambda i: (i, 0)),
        out_shape=jax.ShapeDtypeStruct((_NPAD, Fout), jnp.float32),
    )(xp, W, d0, d1)


def _tc_mid(a0, a1, g, d0, d1, b, W):
    Fin, Fout = W.shape

    def body(a0_ref, a1_ref, g_ref, d0_ref, d1_ref, b_ref, w_ref, o_ref):
        dinv = _dinv_of(d0_ref[...], d1_ref[...])
        h = jnp.maximum(dinv * (a0_ref[...] + a1_ref[...] + g_ref[...]) + b_ref[...], 0.0)
        o_ref[...] = dinv * jnp.dot(h, w_ref[...], preferred_element_type=jnp.float32)

    return pl.pallas_call(
        body,
        grid=(_GRID,),
        in_specs=[
            pl.BlockSpec((_BLK, Fin), lambda i: (i, 0)),
            pl.BlockSpec((_BLK, Fin), lambda i: (i, 0)),
            pl.BlockSpec((_BLK, Fin), lambda i: (i, 0)),
            pl.BlockSpec((_BLK, 1), lambda i: (i, 0)),
            pl.BlockSpec((_BLK, 1), lambda i: (i, 0)),
            pl.BlockSpec((1, Fin), lambda i: (0, 0)),
            pl.BlockSpec((Fin, Fout), lambda i: (0, 0)),
        ],
        out_specs=pl.BlockSpec((_BLK, Fout), lambda i: (i, 0)),
        out_shape=jax.ShapeDtypeStruct((_NPAD, Fout), jnp.float32),
    )(a0, a1, g, d0, d1, b, W)


def _tc_last(a0, a1, g, d0, d1, b):
    F = g.shape[1]

    def body(a0_ref, a1_ref, g_ref, d0_ref, d1_ref, b_ref, o_ref):
        dinv = _dinv_of(d0_ref[...], d1_ref[...])
        o = (dinv * (a0_ref[...] + a1_ref[...] + g_ref[...]) + b_ref[...])[:, :2]
        m = jnp.max(o, axis=1, keepdims=True)
        e = jnp.exp(o - m)
        o_ref[...] = o - m - jnp.log(jnp.sum(e, axis=1, keepdims=True))

    return pl.pallas_call(
        body,
        grid=(_GRID,),
        in_specs=[
            pl.BlockSpec((_BLK, F), lambda i: (i, 0)),
            pl.BlockSpec((_BLK, F), lambda i: (i, 0)),
            pl.BlockSpec((_BLK, F), lambda i: (i, 0)),
            pl.BlockSpec((_BLK, 1), lambda i: (i, 0)),
            pl.BlockSpec((_BLK, 1), lambda i: (i, 0)),
            pl.BlockSpec((1, F), lambda i: (0, 0)),
        ],
        out_specs=pl.BlockSpec((_BLK, 2), lambda i: (i, 0)),
        out_shape=jax.ShapeDtypeStruct((_NPAD, 2), jnp.float32),
    )(a0, a1, g, d0, d1, b)


def kernel(x, edge_index, W1, b1, W2, b2, W3, b3):
    xp = jnp.pad(x, ((0, _NPAD - _N), (0, 0)))
    src = edge_index[0].astype(jnp.int32)
    dst = edge_index[1].astype(jnp.int32)
    pad = _EPAD - _E
    it = jnp.arange(pad, dtype=jnp.int32)
    srcp = jnp.concatenate([src, it % _N]).reshape(-1, _LANES)
    dstp = jnp.concatenate([dst, _N + it % (_NPAD - _N)]).reshape(-1, _LANES)

    degp = _make_deg()(dstp, jnp.zeros((_NPAD,), jnp.float32)).reshape(2, _NPAD)
    d0 = degp[0].reshape(_NPAD, 1)
    d1 = degp[1].reshape(_NPAD, 1)

    g1 = _tc_first(xp, W1, d0, d1)
    a1 = _make_agg(32)(srcp, dstp, g1, jnp.zeros((_NPAD, 32), jnp.float32))
    g2 = _tc_mid(a1[0], a1[1], g1, d0, d1, b1.reshape(1, -1), W2)
    a2 = _make_agg(16)(srcp, dstp, g2, jnp.zeros((_NPAD, 16), jnp.float32))
    W3p = jnp.pad(W3, ((0, 0), (0, 6)))
    b3p = jnp.pad(b3, (0, 6)).reshape(1, 8)
    g3 = _tc_mid(a2[0], a2[1], g2, d0, d1, b2.reshape(1, -1), W3p)
    a3 = _make_agg(8)(srcp, dstp, g3, jnp.zeros((_NPAD, 8), jnp.float32))
    o = _tc_last(a3[0], a3[1], g3, d0, d1, b3p)
    return o[:_N]


# speedup vs baseline: 49.8366x; 1.3222x over previous
import functools

import jax
import jax.numpy as jnp
from jax import lax
from jax.experimental import pallas as pl
from jax.experimental.pallas import tpu as pltpu
from jax.experimental.pallas import tpu_sc as plsc

_N = 10000
_NPAD = 10240
_E = 320000
_LANES = 128
_NTILES = 32
_CPT = 80
_EPAD = _NTILES * _CPT * _LANES
_RPT = _NPAD // 16
_BLK = 512
_GRID = _NPAD // _BLK


def _sc_mesh():
    return plsc.VectorSubcoreMesh(
        core_axis_name="c", subcore_axis_name="s", num_cores=2, num_subcores=16
    )


def _make_agg(F):
    K = 8
    G = _CPT // K

    @functools.partial(
        pl.kernel,
        out_type=jax.ShapeDtypeStruct((2 * _NPAD, F), jnp.float32),
        mesh=_sc_mesh(),
        compiler_params=pltpu.CompilerParams(use_tc_tiling_on_sc=False),
        scratch_types=[
            pltpu.VMEM((_CPT, _LANES), jnp.int32),
            pltpu.VMEM((_CPT, _LANES), jnp.int32),
            pltpu.VMEM((K * _LANES, F), jnp.float32),
            pltpu.VMEM((K * _LANES, F), jnp.float32),
            pltpu.VMEM_SHARED((_NPAD, F), jnp.float32),
            pltpu.SemaphoreType.DMA,
            pltpu.SemaphoreType.DMA,
            pltpu.SemaphoreType.DMA,
            pltpu.SemaphoreType.DMA,
        ],
    )
    def agg(src_hbm, dst_hbm, g_hbm, z_hbm, out_hbm, src_v, dst_v, buf_a, buf_b,
            acc_sh, gsem_a, gsem_b, ssem_a, ssem_b):
        c = lax.axis_index("c")
        s = lax.axis_index("s")
        wid = c * 16 + s
        row0 = s * _RPT
        pltpu.sync_copy(z_hbm.at[pl.ds(row0, _RPT)], acc_sh.at[pl.ds(row0, _RPT)])
        pltpu.sync_copy(src_hbm.at[pl.ds(wid * _CPT, _CPT)], src_v)
        pltpu.sync_copy(dst_hbm.at[pl.ds(wid * _CPT, _CPT)], dst_v)
        plsc.subcore_barrier()

        def gathers(g, buf, sem, issue):
            for b in range(K):
                d = pltpu.make_async_copy(
                    g_hbm.at[src_v.at[g * K + b]], buf.at[pl.ds(b * _LANES, _LANES)], sem
                )
                d.start() if issue else d.wait()

        def scatters(g, buf, sem, issue):
            for b in range(K):
                if issue:
                    pltpu.async_copy(
                        buf.at[pl.ds(b * _LANES, _LANES)],
                        acc_sh.at[dst_v.at[g * K + b]], sem, add=True,
                    )
                else:
                    pltpu.make_async_copy(
                        buf.at[pl.ds(b * _LANES, _LANES)],
                        acc_sh.at[dst_v.at[g * K + b]], sem,
                    ).wait()

        def group(g, buf, gsem, ssem, prefetch):
            gathers(g, buf, gsem, False)
            scatters(g, buf, ssem, True)
            scatters(g, buf, ssem, False)
            if prefetch:
                gathers(g + 2, buf, gsem, True)

        gathers(0, buf_a, gsem_a, True)
        gathers(1, buf_b, gsem_b, True)

        def pair(gp, carry):
            group(2 * gp, buf_a, gsem_a, ssem_a, True)
            group(2 * gp + 1, buf_b, gsem_b, ssem_b, True)
            return carry

        lax.fori_loop(0, G // 2 - 1, pair, 0)
        group(G - 2, buf_a, gsem_a, ssem_a, False)
        group(G - 1, buf_b, gsem_b, ssem_b, False)
        plsc.subcore_barrier()
        pltpu.sync_copy(
            acc_sh.at[pl.ds(row0, _RPT)], out_hbm.at[pl.ds(c * _NPAD + row0, _RPT)]
        )

    return agg


def _make_deg():
    @functools.partial(
        pl.kernel,
        out_type=jax.ShapeDtypeStruct((2 * _NPAD,), jnp.float32),
        mesh=_sc_mesh(),
        scratch_types=[
            pltpu.VMEM((_CPT, _LANES), jnp.int32),
            pltpu.VMEM((_LANES,), jnp.float32),
            pltpu.VMEM_SHARED((_NPAD,), jnp.float32),
            pltpu.SemaphoreType.DMA,
        ],
    )
    def deg(dst_hbm, z_hbm, out_hbm, dst_v, ones_v, acc_sh, sem):
        c = lax.axis_index("c")
        s = lax.axis_index("s")
        wid = c * 16 + s
        row0 = s * _RPT
        pltpu.sync_copy(z_hbm.at[pl.ds(row0, _RPT)], acc_sh.at[pl.ds(row0, _RPT)])
        pltpu.sync_copy(dst_hbm.at[pl.ds(wid * _CPT, _CPT)], dst_v)
        one = jnp.ones((16,), jnp.float32)
        for j in range(_LANES // 16):
            ones_v[pl.ds(j * 16, 16)] = one
        plsc.subcore_barrier()

        def body(i, carry):
            pltpu.async_copy(ones_v, acc_sh.at[dst_v.at[i]], sem, add=True)
            return carry

        lax.fori_loop(0, _CPT, body, 0)

        def drain(i, carry):
            pltpu.make_async_copy(ones_v, acc_sh.at[dst_v.at[i]], sem).wait()
            return carry

        lax.fori_loop(0, _CPT, drain, 0)
        plsc.subcore_barrier()
        pltpu.sync_copy(
            acc_sh.at[pl.ds(row0, _RPT)], out_hbm.at[pl.ds(c * _NPAD + row0, _RPT)]
        )

    return deg


def _dinv_blk(d_ref0, d_ref1):
    deg = d_ref0[...] + d_ref1[...] + 1.0
    return lax.rsqrt(jnp.maximum(deg, 1e-12))[:, None]


def _dspec():
    return [
        pl.BlockSpec((_BLK,), lambda i: (i,)),
        pl.BlockSpec((_BLK,), lambda i: (i + _GRID,)),
    ]


def _aspec(F):
    return [
        pl.BlockSpec((_BLK, F), lambda i: (i, 0)),
        pl.BlockSpec((_BLK, F), lambda i: (i + _GRID, 0)),
    ]


def _tc_first(xp, W, d):
    Fin, Fout = W.shape

    def body(x_ref, w_ref, d0_ref, d1_ref, o_ref):
        dinv = _dinv_blk(d0_ref, d1_ref)
        o_ref[...] = dinv * jnp.dot(
            x_ref[...], w_ref[...], preferred_element_type=jnp.float32
        )

    return pl.pallas_call(
        body,
        grid=(_GRID,),
        in_specs=[
            pl.BlockSpec((_BLK, Fin), lambda i: (i, 0)),
            pl.BlockSpec((Fin, Fout), lambda i: (0, 0)),
        ] + _dspec(),
        out_specs=pl.BlockSpec((_BLK, Fout), lambda i: (i, 0)),
        out_shape=jax.ShapeDtypeStruct((_NPAD, Fout), jnp.float32),
    )(xp, W, d, d)


def _tc_mid(a, g, d, b, W):
    Fin, Fout = W.shape

    def body(a0_ref, a1_ref, g_ref, d0_ref, d1_ref, b_ref, w_ref, o_ref):
        dinv = _dinv_blk(d0_ref, d1_ref)
        h = jnp.maximum(dinv * (a0_ref[...] + a1_ref[...] + g_ref[...]) + b_ref[...], 0.0)
        o_ref[...] = dinv * jnp.dot(h, w_ref[...], preferred_element_type=jnp.float32)

    return pl.pallas_call(
        body,
        grid=(_GRID,),
        in_specs=_aspec(Fin) + [
            pl.BlockSpec((_BLK, Fin), lambda i: (i, 0)),
        ] + _dspec() + [
            pl.BlockSpec((1, Fin), lambda i: (0, 0)),
            pl.BlockSpec((Fin, Fout), lambda i: (0, 0)),
        ],
        out_specs=pl.BlockSpec((_BLK, Fout), lambda i: (i, 0)),
        out_shape=jax.ShapeDtypeStruct((_NPAD, Fout), jnp.float32),
    )(a, a, g, d, d, b, W)


def _tc_last(a, g, d, b):
    F = g.shape[1]

    def body(a0_ref, a1_ref, g_ref, d0_ref, d1_ref, b_ref, o_ref):
        dinv = _dinv_blk(d0_ref, d1_ref)
        o = (dinv * (a0_ref[...] + a1_ref[...] + g_ref[...]) + b_ref[...])[:, :2]
        m = jnp.max(o, axis=1, keepdims=True)
        e = jnp.exp(o - m)
        o_ref[...] = o - m - jnp.log(jnp.sum(e, axis=1, keepdims=True))

    return pl.pallas_call(
        body,
        grid=(_GRID,),
        in_specs=_aspec(F) + [
            pl.BlockSpec((_BLK, F), lambda i: (i, 0)),
        ] + _dspec() + [
            pl.BlockSpec((1, F), lambda i: (0, 0)),
        ],
        out_specs=pl.BlockSpec((_BLK, 2), lambda i: (i, 0)),
        out_shape=jax.ShapeDtypeStruct((_NPAD, 2), jnp.float32),
    )(a, a, g, d, d, b)


def kernel(x, edge_index, W1, b1, W2, b2, W3, b3):
    xp = jnp.pad(x, ((0, _NPAD - _N), (0, 0)))
    src = edge_index[0].astype(jnp.int32)
    dst = edge_index[1].astype(jnp.int32)
    pad = _EPAD - _E
    it = jnp.arange(pad, dtype=jnp.int32)
    srcp = jnp.concatenate([src, it & 127]).reshape(-1, _LANES)
    dstp = jnp.concatenate([dst, _N + (it & 127)]).reshape(-1, _LANES)

    d = _make_deg()(dstp, jnp.zeros((_NPAD,), jnp.float32))

    g1 = _tc_first(xp, W1, d)
    a1 = _make_agg(32)(srcp, dstp, g1, jnp.zeros((_NPAD, 32), jnp.float32))
    g2 = _tc_mid(a1, g1, d, b1.reshape(1, -1), W2)
    a2 = _make_agg(16)(srcp, dstp, g2, jnp.zeros((_NPAD, 16), jnp.float32))
    W3p = jnp.pad(W3, ((0, 0), (0, 6)))
    b3p = jnp.pad(b3, (0, 6)).reshape(1, 8)
    g3 = _tc_mid(a2, g2, d, b2.reshape(1, -1), W3p)
    a3 = _make_agg(8)(srcp, dstp, g3, jnp.zeros((_NPAD, 8), jnp.float32))
    o = _tc_last(a3, g3, d, b3p)
    return o[:_N]


# speedup vs baseline: 57.2104x; 1.1480x over previous
import functools

import jax
import jax.numpy as jnp
from jax import lax
from jax.experimental import pallas as pl
from jax.experimental.pallas import tpu as pltpu
from jax.experimental.pallas import tpu_sc as plsc

_N = 10000
_NPAD = 10240
_E = 320000
_LANES = 128
_NTILES = 32
_CPT = 80
_EPAD = _NTILES * _CPT * _LANES
_RPT = _NPAD // 16
_BLK = 512
_GRID = _NPAD // _BLK


def _sc_mesh():
    return plsc.VectorSubcoreMesh(
        core_axis_name="c", subcore_axis_name="s", num_cores=2, num_subcores=16
    )


def _make_agg(F):
    K = 8
    G = _CPT // K

    @functools.partial(
        pl.kernel,
        out_type=jax.ShapeDtypeStruct((2 * _NPAD, F), jnp.float32),
        mesh=_sc_mesh(),
        compiler_params=pltpu.CompilerParams(use_tc_tiling_on_sc=False),
        scratch_types=[
            pltpu.VMEM((_CPT, _LANES), jnp.int32),
            pltpu.VMEM((_CPT, _LANES), jnp.int32),
            pltpu.VMEM((K * _LANES, F), jnp.float32),
            pltpu.VMEM((K * _LANES, F), jnp.float32),
            pltpu.VMEM_SHARED((_NPAD, F), jnp.float32),
            pltpu.SemaphoreType.DMA,
            pltpu.SemaphoreType.DMA,
            pltpu.SemaphoreType.DMA,
            pltpu.SemaphoreType.DMA,
        ],
    )
    def agg(src_hbm, dst_hbm, g_hbm, z_hbm, out_hbm, src_v, dst_v, buf_a, buf_b,
            acc_sh, gsem_a, gsem_b, ssem_a, ssem_b):
        c = lax.axis_index("c")
        s = lax.axis_index("s")
        wid = c * 16 + s
        row0 = s * _RPT
        pltpu.sync_copy(z_hbm.at[pl.ds(row0, _RPT)], acc_sh.at[pl.ds(row0, _RPT)])
        pltpu.sync_copy(src_hbm.at[pl.ds(wid * _CPT, _CPT)], src_v)
        pltpu.sync_copy(dst_hbm.at[pl.ds(wid * _CPT, _CPT)], dst_v)
        plsc.subcore_barrier()

        def gathers(g, buf, sem, issue):
            for b in range(K):
                d = pltpu.make_async_copy(
                    g_hbm.at[src_v.at[g * K + b]], buf.at[pl.ds(b * _LANES, _LANES)], sem
                )
                d.start() if issue else d.wait()

        def scatters(g, buf, sem, issue):
            for b in range(K):
                if issue:
                    pltpu.async_copy(
                        buf.at[pl.ds(b * _LANES, _LANES)],
                        acc_sh.at[dst_v.at[g * K + b]], sem, add=True,
                    )
                else:
                    pltpu.make_async_copy(
                        buf.at[pl.ds(b * _LANES, _LANES)],
                        acc_sh.at[dst_v.at[g * K + b]], sem,
                    ).wait()

        def group(g, buf, gsem, ssem, prefetch):
            gathers(g, buf, gsem, False)
            scatters(g, buf, ssem, True)
            scatters(g, buf, ssem, False)
            if prefetch:
                gathers(g + 2, buf, gsem, True)

        gathers(0, buf_a, gsem_a, True)
        gathers(1, buf_b, gsem_b, True)

        def pair(gp, carry):
            group(2 * gp, buf_a, gsem_a, ssem_a, True)
            group(2 * gp + 1, buf_b, gsem_b, ssem_b, True)
            return carry

        lax.fori_loop(0, G // 2 - 1, pair, 0)
        group(G - 2, buf_a, gsem_a, ssem_a, False)
        group(G - 1, buf_b, gsem_b, ssem_b, False)
        plsc.subcore_barrier()
        pltpu.sync_copy(
            acc_sh.at[pl.ds(row0, _RPT)], out_hbm.at[pl.ds(c * _NPAD + row0, _RPT)]
        )

    return agg


def _make_deg():
    @functools.partial(
        pl.kernel,
        out_type=jax.ShapeDtypeStruct((2 * _NPAD,), jnp.float32),
        mesh=_sc_mesh(),
        scratch_types=[
            pltpu.VMEM((_CPT, _LANES), jnp.int32),
            pltpu.VMEM((_LANES,), jnp.float32),
            pltpu.VMEM_SHARED((_NPAD,), jnp.float32),
            pltpu.SemaphoreType.DMA,
        ],
    )
    def deg(dst_hbm, z_hbm, out_hbm, dst_v, ones_v, acc_sh, sem):
        c = lax.axis_index("c")
        s = lax.axis_index("s")
        wid = c * 16 + s
        row0 = s * _RPT
        pltpu.sync_copy(z_hbm.at[pl.ds(row0, _RPT)], acc_sh.at[pl.ds(row0, _RPT)])
        pltpu.sync_copy(dst_hbm.at[pl.ds(wid * _CPT, _CPT)], dst_v)
        one = jnp.ones((16,), jnp.float32)
        for j in range(_LANES // 16):
            ones_v[pl.ds(j * 16, 16)] = one
        plsc.subcore_barrier()

        def body(i, carry):
            pltpu.async_copy(ones_v, acc_sh.at[dst_v.at[i]], sem, add=True)
            return carry

        lax.fori_loop(0, _CPT, body, 0)

        def drain(i, carry):
            pltpu.make_async_copy(ones_v, acc_sh.at[dst_v.at[i]], sem).wait()
            return carry

        lax.fori_loop(0, _CPT, drain, 0)
        plsc.subcore_barrier()
        pltpu.sync_copy(
            acc_sh.at[pl.ds(row0, _RPT)], out_hbm.at[pl.ds(c * _NPAD + row0, _RPT)]
        )

    return deg


def _tc_dinv_packed(d):
    d0, d1 = d[:_NPAD], d[_NPAD:]
    views = [
        (d0.reshape(_NPAD // 4, 4), d1.reshape(_NPAD // 4, 4)),
        (d0.reshape(_NPAD // 8, 8), d1.reshape(_NPAD // 8, 8)),
        (d0.reshape(_NPAD // 16, 16), d1.reshape(_NPAD // 16, 16)),
    ]
    mats = [
        jnp.kron(jnp.eye(4, dtype=jnp.float32), jnp.ones((1, 32), jnp.float32)),
        jnp.kron(jnp.eye(8, dtype=jnp.float32), jnp.ones((1, 16), jnp.float32)),
        jnp.kron(jnp.eye(16, dtype=jnp.float32), jnp.ones((1, 8), jnp.float32)),
    ]

    def body(a4, b4, a8, b8, a16, b16, s4, s8, s16, o32, o16, o8):
        for a, b, s, o in ((a4, b4, s4, o32), (a8, b8, s8, o16), (a16, b16, s16, o8)):
            dv = lax.rsqrt(jnp.maximum(a[...] + b[...] + 1.0, 1e-12))
            o[...] = jnp.dot(dv, s[...], preferred_element_type=jnp.float32)

    m = 4
    r = _NPAD // 4 // m
    return pl.pallas_call(
        body,
        grid=(m,),
        in_specs=[
            pl.BlockSpec((r, 4), lambda i: (i, 0)),
            pl.BlockSpec((r, 4), lambda i: (i, 0)),
            pl.BlockSpec((r // 2, 8), lambda i: (i, 0)),
            pl.BlockSpec((r // 2, 8), lambda i: (i, 0)),
            pl.BlockSpec((r // 4, 16), lambda i: (i, 0)),
            pl.BlockSpec((r // 4, 16), lambda i: (i, 0)),
            pl.BlockSpec((4, 128), lambda i: (0, 0)),
            pl.BlockSpec((8, 128), lambda i: (0, 0)),
            pl.BlockSpec((16, 128), lambda i: (0, 0)),
        ],
        out_specs=[
            pl.BlockSpec((r, 128), lambda i: (i, 0)),
            pl.BlockSpec((r // 2, 128), lambda i: (i, 0)),
            pl.BlockSpec((r // 4, 128), lambda i: (i, 0)),
        ],
        out_shape=[
            jax.ShapeDtypeStruct((_NPAD // 4, 128), jnp.float32),
            jax.ShapeDtypeStruct((_NPAD // 8, 128), jnp.float32),
            jax.ShapeDtypeStruct((_NPAD // 16, 128), jnp.float32),
        ],
    )(views[0][0], views[0][1], views[1][0], views[1][1], views[2][0], views[2][1],
      *mats)


def _tc_first(xp, W1, dp32):
    wt = jnp.kron(jnp.eye(4, dtype=jnp.float32), W1)
    xv = xp.reshape(_NPAD // 4, 512)

    def body(x_ref, w_ref, dp_ref, o_ref):
        o_ref[...] = dp_ref[...] * jnp.dot(
            x_ref[...], w_ref[...], preferred_element_type=jnp.float32
        )

    m = 20
    r = _NPAD // 4 // m
    return pl.pallas_call(
        body,
        grid=(m,),
        in_specs=[
            pl.BlockSpec((r, 512), lambda i: (i, 0)),
            pl.BlockSpec((512, 128), lambda i: (0, 0)),
            pl.BlockSpec((r, 128), lambda i: (i, 0)),
        ],
        out_specs=pl.BlockSpec((r, 128), lambda i: (i, 0)),
        out_shape=jax.ShapeDtypeStruct((_NPAD // 4, 128), jnp.float32),
    )(xv, wt, dp32)


def _tc_h(a_pk, g_pk, dp, bt):
    rows = g_pk.shape[0]

    def body(a0_ref, a1_ref, g_ref, dp_ref, b_ref, o_ref):
        o_ref[...] = jnp.maximum(
            dp_ref[...] * (a0_ref[...] + a1_ref[...] + g_ref[...]) + b_ref[...], 0.0
        )

    m = 10
    r = rows // m
    return pl.pallas_call(
        body,
        grid=(m,),
        in_specs=[
            pl.BlockSpec((r, 128), lambda i: (i, 0)),
            pl.BlockSpec((r, 128), lambda i: (i + m, 0)),
            pl.BlockSpec((r, 128), lambda i: (i, 0)),
            pl.BlockSpec((r, 128), lambda i: (i, 0)),
            pl.BlockSpec((1, 128), lambda i: (0, 0)),
        ],
        out_specs=pl.BlockSpec((r, 128), lambda i: (i, 0)),
        out_shape=jax.ShapeDtypeStruct((rows, 128), jnp.float32),
    )(a_pk, a_pk, g_pk, dp, bt)


def _tc_mm(h_pk, W, k, dp_out):
    wt = jnp.kron(jnp.eye(k, dtype=jnp.float32), W)
    hv = h_pk.reshape(h_pk.shape[0] // 2, 256)
    rows = hv.shape[0]

    def body(h_ref, w_ref, dp_ref, o_ref):
        o_ref[...] = dp_ref[...] * jnp.dot(
            h_ref[...], w_ref[...], preferred_element_type=jnp.float32
        )

    m = 5
    r = rows // m
    return pl.pallas_call(
        body,
        grid=(m,),
        in_specs=[
            pl.BlockSpec((r, 256), lambda i: (i, 0)),
            pl.BlockSpec((256, 128), lambda i: (0, 0)),
            pl.BlockSpec((r, 128), lambda i: (i, 0)),
        ],
        out_specs=pl.BlockSpec((r, 128), lambda i: (i, 0)),
        out_shape=jax.ShapeDtypeStruct((rows, 128), jnp.float32),
    )(hv, wt, dp_out)


def _tc_last(a_pk, g_pk, dp8, bt, pswap):
    rows = g_pk.shape[0]

    def body(a0_ref, a1_ref, g_ref, dp_ref, b_ref, p_ref, o_ref):
        o = dp_ref[...] * (a0_ref[...] + a1_ref[...] + g_ref[...]) + b_ref[...]
        partner = jnp.dot(o, p_ref[...], preferred_element_type=jnp.float32)
        mx = jnp.maximum(o, partner)
        ls = mx + jnp.log(jnp.exp(o - mx) + jnp.exp(partner - mx))
        o_ref[...] = o - ls

    m = 5
    r = rows // m
    return pl.pallas_call(
        body,
        grid=(m,),
        in_specs=[
            pl.BlockSpec((r, 128), lambda i: (i, 0)),
            pl.BlockSpec((r, 128), lambda i: (i + m, 0)),
            pl.BlockSpec((r, 128), lambda i: (i, 0)),
            pl.BlockSpec((r, 128), lambda i: (i, 0)),
            pl.BlockSpec((1, 128), lambda i: (0, 0)),
            pl.BlockSpec((128, 128), lambda i: (0, 0)),
        ],
        out_specs=pl.BlockSpec((r, 128), lambda i: (i, 0)),
        out_shape=jax.ShapeDtypeStruct((rows, 128), jnp.float32),
    )(a_pk, a_pk, g_pk, dp8, bt, pswap)


def kernel(x, edge_index, W1, b1, W2, b2, W3, b3):
    xp = jnp.pad(x, ((0, _NPAD - _N), (0, 0)))
    src = edge_index[0].astype(jnp.int32)
    dst = edge_index[1].astype(jnp.int32)
    pad = _EPAD - _E
    it = jnp.arange(pad, dtype=jnp.int32)
    srcp = jnp.concatenate([src, it & 127]).reshape(-1, _LANES)
    dstp = jnp.concatenate([dst, _N + (it & 127)]).reshape(-1, _LANES)

    d = _make_deg()(dstp, jnp.zeros((_NPAD,), jnp.float32))
    dp32, dp16, dp8 = _tc_dinv_packed(d)

    g1 = _tc_first(xp, W1, dp32)
    a1 = _make_agg(32)(srcp, dstp, g1.reshape(_NPAD, 32),
                       jnp.zeros((_NPAD, 32), jnp.float32))
    h2 = _tc_h(a1.reshape(_NPAD // 2, 128), g1, dp32,
               jnp.tile(b1, 4).reshape(1, 128))
    g2 = _tc_mm(h2, W2, 8, dp16)
    a2 = _make_agg(16)(srcp, dstp, g2.reshape(_NPAD, 16),
                       jnp.zeros((_NPAD, 16), jnp.float32))
    W3p = jnp.pad(W3, ((0, 0), (0, 6)))
    h3 = _tc_h(a2.reshape(_NPAD // 4, 128), g2, dp16,
               jnp.tile(b2, 8).reshape(1, 128))
    g3 = _tc_mm(h3, W3p, 16, dp8)
    a3 = _make_agg(8)(srcp, dstp, g3.reshape(_NPAD, 8),
                      jnp.zeros((_NPAD, 8), jnp.float32))
    b3p = jnp.tile(jnp.pad(b3, (0, 6)), 16).reshape(1, 128)
    pswap = jnp.kron(
        jnp.eye(16, dtype=jnp.float32),
        jnp.eye(8, dtype=jnp.float32)[jnp.array([1, 0, 2, 3, 4, 5, 6, 7])],
    )
    o = _tc_last(a3.reshape(_NPAD // 8, 128), g3, dp8, b3p, pswap)
    return o.reshape(_NPAD, 8)[:_N, :2]
